# baseline (speedup 1.0000x reference)
import functools
import math

import jax
import jax.numpy as jnp
from jax import lax
from jax.experimental import pallas as pl
from jax.experimental.pallas import tpu as pltpu
from jax.experimental.pallas import tpu_sc as plsc

_M = 4
_COS_M = math.cos(_M)
_SIN_M = math.sin(_M)

_LN = 16
_TILE = 128


def _sc_gather_tiles(xt, lbl_i32):
    c, b = xt.shape
    info = plsc.get_sparse_core_info()
    nw = info.num_cores * info.num_subcores
    rpw = b // nw
    assert rpw * (nw // 4) * 4 == b and (rpw * 4) % _TILE == 0

    mesh = plsc.VectorSubcoreMesh(core_axis_name="c", subcore_axis_name="s")

    @functools.partial(
        pl.kernel,
        mesh=mesh,
        out_type=jax.ShapeDtypeStruct((b, 8, _TILE), jnp.float32),
        scratch_types=[
            pltpu.VMEM((_TILE,), jnp.int32),
            pltpu.SemaphoreType.DMA,
        ],
        compiler_params=pltpu.CompilerParams(needs_layout_passes=False),
    )
    def k(xt_hbm, lbl_hbm, tiles_hbm, lblv, semg):
        wid = lax.axis_index("s") * info.num_cores + lax.axis_index("c")
        i0 = wid * rpw
        col0 = (wid // 4) * _TILE
        pltpu.sync_copy(lbl_hbm.at[pl.ds((wid // 4) * _TILE, _TILE)], lblv)
        loff = (wid % 4) * rpw
        lane_iota = lax.broadcasted_iota(jnp.int32, (_LN,), 0)
        gds = []
        for t in range(rpw):
            vec = lblv[pl.ds(loff + (t // _LN) * _LN, _LN)]
            sel = jnp.where(lane_iota == (t % _LN), vec, 0)
            s = jnp.max(sel)
            gds.append(pltpu.async_copy(
                xt_hbm.at[pl.ds((s // 8) * 8, 8), pl.ds(col0, _TILE)],
                tiles_hbm.at[i0 + t], semg))
        for d in gds:
            d.wait()

    return k(xt, lbl_i32)


def _tc_sum(x_ref, out_ref, acc_ref):
    j = pl.program_id(0)
    nj = pl.num_programs(0)

    @pl.when(j == 0)
    def _init():
        acc_ref[0, 0] = 0.0

    x = x_ref[...]
    acc_ref[0, 0] += jnp.sum(x * x)

    @pl.when(j == nj - 1)
    def _out():
        out_ref[0, 0, 0] = acc_ref[0, 0]


def _tc_fin(part_ref, tiles_ref, lbl_ref, out_ref, *, n):
    total = part_ref[0, 0, 0]
    tiles = tiles_ref[...]
    lbl = lbl_ref[...]
    t0i = jax.lax.broadcasted_iota(jnp.int32, tiles.shape, 0)
    t1i = jax.lax.broadcasted_iota(jnp.int32, tiles.shape, 1)
    t2i = jax.lax.broadcasted_iota(jnp.int32, tiles.shape, 2)
    m = (t1i == lbl.reshape(-1, 1, 1) % 8) & (t2i == t0i % _TILE)
    v = jnp.sum(jnp.where(m, tiles, 0.0), axis=(1, 2)).reshape(-1, 1)
    phi = v * _COS_M - jnp.sqrt(jnp.maximum(1.0 - v * v, 0.0)) * _SIN_M
    corr = jnp.sum(phi * phi - v * v)
    out_ref[0, 0, 0] = (total + corr) * (_M * _M / n)


def kernel(input, label):
    b, c = input.shape
    xt = input.T
    lbl = label.astype(jnp.int32)

    tiles = _sc_gather_tiles(xt, lbl)

    br = 4000
    assert c % br == 0 and br % 8 == 0
    part = pl.pallas_call(
        _tc_sum,
        grid=(c // br,),
        in_specs=[pl.BlockSpec((br, b), lambda j: (j, 0))],
        out_specs=pl.BlockSpec((1, 1, 1), lambda j: (0, 0, 0),
                               memory_space=pltpu.SMEM),
        out_shape=jax.ShapeDtypeStruct((1, 1, 1), jnp.float32),
        scratch_shapes=[pltpu.SMEM((1, 1), jnp.float32)],
    )(xt)

    out = pl.pallas_call(
        functools.partial(_tc_fin, n=b * c),
        out_specs=pl.BlockSpec((1, 1, 1), lambda: (0, 0, 0),
                               memory_space=pltpu.SMEM),
        out_shape=jax.ShapeDtypeStruct((1, 1, 1), jnp.float32),
    )(part, tiles, lbl.reshape(b, 1))
    return out.reshape(())


# speedup vs baseline: 1.0029x; 1.0029x over previous
import functools
import math

import jax
import jax.numpy as jnp
from jax import lax
from jax.experimental import pallas as pl
from jax.experimental.pallas import tpu as pltpu
from jax.experimental.pallas import tpu_sc as plsc

_M = 4
_COS_M = math.cos(_M)
_SIN_M = math.sin(_M)

_LN = 16
_TILE = 128


def _sc_gather_tiles(xt, lbl_i32):
    c, b = xt.shape
    info = plsc.get_sparse_core_info()
    nw = info.num_cores * info.num_subcores
    rpw = b // nw
    assert rpw * (nw // 4) * 4 == b and (rpw * 4) % _TILE == 0

    mesh = plsc.VectorSubcoreMesh(core_axis_name="c", subcore_axis_name="s")

    @functools.partial(
        pl.kernel,
        mesh=mesh,
        out_type=jax.ShapeDtypeStruct((b, 8, _TILE), jnp.float32),
        scratch_types=[
            pltpu.VMEM((_TILE,), jnp.int32),
            pltpu.SemaphoreType.DMA,
        ],
        compiler_params=pltpu.CompilerParams(needs_layout_passes=False),
    )
    def k(xt_hbm, lbl_hbm, tiles_hbm, lblv, semg):
        wid = lax.axis_index("s") * info.num_cores + lax.axis_index("c")
        i0 = wid * rpw
        col0 = (wid // 4) * _TILE
        pltpu.sync_copy(lbl_hbm.at[pl.ds((wid // 4) * _TILE, _TILE)], lblv)
        loff = (wid % 4) * rpw
        lane_iota = lax.broadcasted_iota(jnp.int32, (_LN,), 0)
        gds = []
        for t in range(rpw):
            vec = lblv[pl.ds(loff + (t // _LN) * _LN, _LN)]
            sel = jnp.where(lane_iota == (t % _LN), vec, 0)
            s = jnp.max(sel)
            gds.append(pltpu.async_copy(
                xt_hbm.at[pl.ds((s // 8) * 8, 8), pl.ds(col0, _TILE)],
                tiles_hbm.at[i0 + t], semg))
        for d in gds:
            d.wait()

    return k(xt, lbl_i32)


def _tc_sum(x_ref, out_ref, acc_ref):
    j = pl.program_id(0)
    nj = pl.num_programs(0)

    @pl.when(j == 0)
    def _init():
        acc_ref[0, 0] = 0.0

    x = x_ref[...]
    acc_ref[0, 0] += jnp.sum(x * x)

    @pl.when(j == nj - 1)
    def _out():
        out_ref[0, 0, 0] = acc_ref[0, 0]


def _tc_fin(part_ref, tiles_ref, lbl_ref, out_ref, *, n):
    total = part_ref[0, 0, 0]
    tiles = tiles_ref[...]
    lbl = lbl_ref[...]
    t0i = jax.lax.broadcasted_iota(jnp.int32, tiles.shape, 0)
    t1i = jax.lax.broadcasted_iota(jnp.int32, tiles.shape, 1)
    t2i = jax.lax.broadcasted_iota(jnp.int32, tiles.shape, 2)
    m = (t1i == lbl.reshape(-1, 1, 1) % 8) & (t2i == t0i % _TILE)
    v = jnp.sum(jnp.where(m, tiles, 0.0), axis=(1, 2)).reshape(-1, 1)
    phi = v * _COS_M - jnp.sqrt(jnp.maximum(1.0 - v * v, 0.0)) * _SIN_M
    corr = jnp.sum(phi * phi - v * v)
    out_ref[0, 0, 0] = (total + corr) * (_M * _M / n)


def kernel(input, label):
    b, c = input.shape
    xt = input.T
    lbl = label.astype(jnp.int32)

    tiles = _sc_gather_tiles(xt, lbl)

    br = 5000
    assert c % br == 0 and br % 8 == 0
    part = pl.pallas_call(
        _tc_sum,
        grid=(c // br,),
        in_specs=[pl.BlockSpec((br, b), lambda j: (j, 0))],
        out_specs=pl.BlockSpec((1, 1, 1), lambda j: (0, 0, 0),
                               memory_space=pltpu.SMEM),
        out_shape=jax.ShapeDtypeStruct((1, 1, 1), jnp.float32),
        scratch_shapes=[pltpu.SMEM((1, 1), jnp.float32)],
    )(xt)

    out = pl.pallas_call(
        functools.partial(_tc_fin, n=b * c),
        out_specs=pl.BlockSpec((1, 1, 1), lambda: (0, 0, 0),
                               memory_space=pltpu.SMEM),
        out_shape=jax.ShapeDtypeStruct((1, 1, 1), jnp.float32),
    )(part, tiles, lbl.reshape(b, 1))
    return out.reshape(())
